# baseline (speedup 1.0000x reference)
import functools

import jax
import jax.numpy as jnp
from jax import lax
from jax.experimental import pallas as pl
from jax.experimental.pallas import tpu as pltpu
from jax.experimental.pallas import tpu_sc as plsc

_NC = 2
_NS = 16
_NW = _NC * _NS


def _field_embedding_lookup(xTf, Wp, *, B, F, V, E):
    assert E == _NW
    n_sl = B // 16

    mesh = plsc.VectorSubcoreMesh(
        core_axis_name="c", subcore_axis_name="s",
        num_cores=_NC, num_subcores=_NS)

    @functools.partial(
        pl.kernel,
        out_type=jax.ShapeDtypeStruct((F, E, B), jnp.float32),
        mesh=mesh,
        scratch_types=[
            pltpu.VMEM((V,), jnp.float32),
            pltpu.VMEM((B,), jnp.float32),
            pltpu.VMEM_SHARED((2, B), jnp.float32),
            pltpu.SemaphoreType.DMA,
        ],
        compiler_params=pltpu.CompilerParams(
            use_tc_tiling_on_sc=True, needs_layout_passes=False),
    )
    def k(xTf_hbm, Wp_hbm, out_hbm, plane_v, buf_v, xsp, sem_w):
        sid = lax.axis_index("s")
        e = sid * _NC + lax.axis_index("c")

        def stage_idx(f):
            @pl.when(sid == f % _NS)
            def _():
                pltpu.sync_copy(xTf_hbm.at[f], xsp.at[f % 2])

        stage_idx(0)
        pltpu.sync_copy(Wp_hbm.at[0, e], plane_v)
        plsc.subcore_barrier()
        pltpu.sync_copy(xsp.at[0], buf_v)
        for f in range(F):
            @plsc.parallel_loop(0, n_sl, unroll=8)
            def body(i):
                s = pl.ds(i * 16, 16)
                iv = plsc.bitcast(buf_v[s], jnp.int32)
                buf_v[s] = plsc.load_gather(plane_v, [iv])

            wr = pltpu.async_copy(buf_v, out_hbm.at[f, e], sem_w)
            if f + 1 < F:
                stage_idx(f + 1)
                pltpu.sync_copy(Wp_hbm.at[f + 1, e], plane_v)
            wr.wait()
            if f + 1 < F:
                plsc.subcore_barrier()
                pltpu.sync_copy(xsp.at[(f + 1) % 2], buf_v)

    return k(xTf, Wp)


def kernel(x, W):
    B, F = x.shape
    _, V, E = W.shape
    xTf = lax.bitcast_convert_type(x.T, jnp.float32)
    Wp = jnp.transpose(W, (0, 2, 1))
    out_p = _field_embedding_lookup(xTf, Wp, B=B, F=F, V=V, E=E)
    return jnp.transpose(out_p, (2, 0, 1))


# speedup vs baseline: 1.1726x; 1.1726x over previous
import functools

import jax
import jax.numpy as jnp
from jax import lax
from jax.experimental import pallas as pl
from jax.experimental.pallas import tpu as pltpu
from jax.experimental.pallas import tpu_sc as plsc

_NC = 2
_NS = 16
_NW = _NC * _NS


def _field_embedding_lookup(xTf, Wp, *, B, F, V, E):
    assert E == _NW
    n_sl = B // 16

    mesh = plsc.VectorSubcoreMesh(
        core_axis_name="c", subcore_axis_name="s",
        num_cores=_NC, num_subcores=_NS)

    @functools.partial(
        pl.kernel,
        out_type=jax.ShapeDtypeStruct((F, E, B), jnp.float32),
        mesh=mesh,
        scratch_types=[
            pltpu.VMEM((V,), jnp.float32),
            pltpu.VMEM((B,), jnp.float32),
            pltpu.VMEM_SHARED((2, B), jnp.float32),
            pltpu.SemaphoreType.DMA,
            pltpu.SemaphoreType.DMA,
        ],
        compiler_params=pltpu.CompilerParams(
            use_tc_tiling_on_sc=True, needs_layout_passes=False),
    )
    def k(xTf_hbm, Wp_hbm, out_hbm, plane_v, buf_v, xsp, sem_w, sem_p):
        sid = lax.axis_index("s")
        e = sid * _NC + lax.axis_index("c")

        def stage_idx(f):
            @pl.when(sid == f % _NS)
            def _():
                pltpu.sync_copy(xTf_hbm.at[f], xsp.at[f % 2])

        stage_idx(0)
        pltpu.sync_copy(Wp_hbm.at[0, e], plane_v)
        plsc.subcore_barrier()
        pltpu.sync_copy(xsp.at[0], buf_v)
        for f in range(F):
            @plsc.parallel_loop(0, n_sl, unroll=16)
            def body(i):
                s = pl.ds(i * 16, 16)
                iv = plsc.bitcast(buf_v[s], jnp.int32)
                buf_v[s] = plsc.load_gather(plane_v, [iv])

            wr = pltpu.async_copy(buf_v, out_hbm.at[f, e], sem_w)
            if f + 1 < F:
                stage_idx(f + 1)
                pln = pltpu.async_copy(Wp_hbm.at[f + 1, e], plane_v, sem_p)
            wr.wait()
            if f + 1 < F:
                plsc.subcore_barrier()
                pltpu.sync_copy(xsp.at[(f + 1) % 2], buf_v)
                pln.wait()

    return k(xTf, Wp)


def kernel(x, W):
    B, F = x.shape
    _, V, E = W.shape
    xTf = lax.bitcast_convert_type(x.T, jnp.float32)
    Wp = jnp.transpose(W, (0, 2, 1))
    out_p = _field_embedding_lookup(xTf, Wp, B=B, F=F, V=V, E=E)
    return jnp.transpose(out_p, (2, 0, 1))
